# Initial kernel scaffold; baseline (speedup 1.0000x reference)
import functools

import jax
import jax.numpy as jnp
from jax import lax
from jax.experimental import pallas as pl
from jax.experimental.pallas import tpu as pltpu
from jax.experimental.pallas import tpu_sc as plsc

LANES = 16


@functools.lru_cache(maxsize=None)
def _build(B, H, V, D, NC, NS):
    NW = NC * NS
    b_per_w = B // NW
    CH = 128
    n_chunks = b_per_w // CH
    GB = CH * H // 128

    mesh = plsc.VectorSubcoreMesh(core_axis_name="c", subcore_axis_name="s")

    @functools.partial(
        pl.kernel,
        mesh=mesh,
        out_type=jax.ShapeDtypeStruct((B, D), jnp.float32),
        scratch_types=[
            pltpu.VMEM((GB, 128), jnp.int32),
            pltpu.VMEM((CH * H, D), jnp.float32),
            pltpu.VMEM((CH, D), jnp.float32),
            pltpu.SemaphoreType.DMA,
        ],
    )
    def emb(idx_hbm, w_hbm, out_hbm, idx_v, rows_v, out_v, sem):
        wid = lax.axis_index("s") * NC + lax.axis_index("c")
        base_b = wid * b_per_w

        def reduce_row(i, _):
            r0 = i * H
            a0 = rows_v[r0, pl.ds(0, LANES)]
            a1 = rows_v[r0, pl.ds(LANES, LANES)]
            for h in range(1, H):
                a0 = a0 + rows_v[r0 + h, pl.ds(0, LANES)]
                a1 = a1 + rows_v[r0 + h, pl.ds(LANES, LANES)]
            out_v[i, pl.ds(0, LANES)] = a0
            out_v[i, pl.ds(LANES, LANES)] = a1
            return 0

        for c in range(n_chunks):
            fb = base_b + c * CH
            g0 = fb * H // 128
            pltpu.sync_copy(idx_hbm.at[pl.ds(g0, GB)], idx_v)
            copies = [
                pltpu.async_copy(
                    w_hbm.at[idx_v.at[g]],
                    rows_v.at[pl.ds(g * 128, 128)],
                    sem,
                )
                for g in range(GB)
            ]
            for cp in copies:
                cp.wait()
            lax.fori_loop(0, CH, reduce_row, 0)
            pltpu.sync_copy(out_v, out_hbm.at[pl.ds(fb, CH)])

    return emb


def kernel(input, weight):
    B, H = input.shape
    V, D = weight.shape
    info = plsc.get_sparse_core_info()
    emb = _build(B, H, V, D, info.num_cores, info.num_subcores)
    idx2d = input.reshape(B * H // 128, 128)
    return emb(idx2d, weight)


# speedup vs baseline: 8.8026x; 8.8026x over previous
import functools

import jax
import jax.numpy as jnp
from jax import lax
from jax.experimental import pallas as pl
from jax.experimental.pallas import tpu as pltpu
from jax.experimental.pallas import tpu_sc as plsc

LANES = 16


@functools.lru_cache(maxsize=None)
def _build(B, H, V, D, NC, NS):
    NW = NC * NS
    b_per_w = B // NW
    CH = 128
    n_chunks = b_per_w // CH
    GB = CH * H // 128

    mesh = plsc.VectorSubcoreMesh(core_axis_name="c", subcore_axis_name="s")

    @functools.partial(
        pl.kernel,
        mesh=mesh,
        out_type=jax.ShapeDtypeStruct((B, D), jnp.float32),
        scratch_types=[
            pltpu.VMEM((n_chunks * GB, 128), jnp.int32),
            pltpu.VMEM((CH * H, D), jnp.float32),
            pltpu.VMEM((CH, D), jnp.float32),
            pltpu.SemaphoreType.DMA,
        ],
        compiler_params=pltpu.CompilerParams(use_tc_tiling_on_sc=False),
    )
    def emb(idx_hbm, w_hbm, out_hbm, idx_v, rows_v, out_v, sem):
        wid = lax.axis_index("s") * NC + lax.axis_index("c")
        base_b = wid * b_per_w
        g0 = pl.multiple_of(base_b * H // 128, 8)
        pltpu.sync_copy(idx_hbm.at[pl.ds(g0, n_chunks * GB)], idx_v)

        def reduce_row(i, _):
            r0 = i * H
            a0 = rows_v[r0, pl.ds(0, LANES)]
            a1 = rows_v[r0, pl.ds(LANES, LANES)]
            for h in range(1, H):
                a0 = a0 + rows_v[r0 + h, pl.ds(0, LANES)]
                a1 = a1 + rows_v[r0 + h, pl.ds(LANES, LANES)]
            out_v[i, pl.ds(0, LANES)] = a0
            out_v[i, pl.ds(LANES, LANES)] = a1
            return 0

        for c in range(n_chunks):
            fb = base_b + c * CH
            copies = [
                pltpu.async_copy(
                    w_hbm.at[idx_v.at[c * GB + g]],
                    rows_v.at[pl.ds(g * 128, 128)],
                    sem,
                )
                for g in range(GB)
            ]
            for cp in copies:
                cp.wait()
            lax.fori_loop(0, CH, reduce_row, 0)
            pltpu.sync_copy(out_v, out_hbm.at[pl.ds(fb, CH)])

    return emb


def kernel(input, weight):
    B, H = input.shape
    V, D = weight.shape
    info = plsc.get_sparse_core_info()
    emb = _build(B, H, V, D, info.num_cores, info.num_subcores)
    idx2d = input.reshape(B * H // 128, 128)
    return emb(idx2d, weight)


# speedup vs baseline: 9.6349x; 1.0945x over previous
import functools

import jax
import jax.numpy as jnp
from jax import lax
from jax.experimental import pallas as pl
from jax.experimental.pallas import tpu as pltpu
from jax.experimental.pallas import tpu_sc as plsc

LANES = 16


@functools.lru_cache(maxsize=None)
def _build(B, H, V, D, NC, NS):
    NW = NC * NS
    b_per_w = B // NW
    CH = 64
    n_chunks = b_per_w // CH
    CR = CH * H

    mesh = plsc.VectorSubcoreMesh(core_axis_name="c", subcore_axis_name="s")

    @functools.partial(
        pl.kernel,
        mesh=mesh,
        out_type=jax.ShapeDtypeStruct((B, D), jnp.float32),
        scratch_types=[
            pltpu.VMEM((b_per_w * H,), jnp.int32),
            pltpu.VMEM((CR, D), jnp.float32),
            pltpu.VMEM((CR, D), jnp.float32),
            pltpu.VMEM((CH, D), jnp.float32),
            pltpu.SemaphoreType.DMA,
            pltpu.SemaphoreType.DMA,
        ],
        compiler_params=pltpu.CompilerParams(use_tc_tiling_on_sc=False),
    )
    def emb(idx_hbm, w_hbm, out_hbm, idx_v, rows0, rows1, out_v, sem0, sem1):
        wid = lax.axis_index("s") * NC + lax.axis_index("c")
        base_b = wid * b_per_w
        i0 = pl.multiple_of(base_b * H, 8)
        pltpu.sync_copy(idx_hbm.at[pl.ds(i0, b_per_w * H)], idx_v)

        rows = (rows0, rows1)
        sems = (sem0, sem1)

        def fire(c):
            return pltpu.async_copy(
                w_hbm.at[idx_v.at[pl.ds(c * CR, CR)]],
                rows[c % 2],
                sems[c % 2],
            )

        pending = fire(0)
        for c in range(n_chunks):
            nxt = fire(c + 1) if c + 1 < n_chunks else None
            pending.wait()
            rv = rows[c % 2]

            @plsc.parallel_loop(0, CH)
            def reduce_row(i):
                r0 = i * H
                a0 = rv[r0, pl.ds(0, LANES)]
                a1 = rv[r0, pl.ds(LANES, LANES)]
                for h in range(1, H):
                    a0 = a0 + rv[r0 + h, pl.ds(0, LANES)]
                    a1 = a1 + rv[r0 + h, pl.ds(LANES, LANES)]
                out_v[i, pl.ds(0, LANES)] = a0
                out_v[i, pl.ds(LANES, LANES)] = a1

            pltpu.sync_copy(out_v, out_hbm.at[pl.ds(base_b + c * CH, CH)])
            pending = nxt

    return emb


def kernel(input, weight):
    B, H = input.shape
    V, D = weight.shape
    info = plsc.get_sparse_core_info()
    emb = _build(B, H, V, D, info.num_cores, info.num_subcores)
    return emb(input.reshape(B * H), weight)
